# baseline (speedup 1.0000x reference)
import jax
import jax.numpy as jnp
from jax.experimental import pallas as pl
from jax.experimental.pallas import tpu as pltpu

N = 8192
BM = 128
NBLK = N // BM
D = 6
NSTEP = 2 * 4 * NBLK

_DOT = dict(precision=jax.lax.Precision.DEFAULT,
            preferred_element_type=jnp.float32)
_BF = jnp.bfloat16


def _kernel(x_ref, a1_ref, p1_ref, a2_ref, p2_ref,
            w1_ref, b1_ref, w2_ref, b2_ref, wl_ref, bl_ref,
            wf_ref, bf_ref,
            lsm_f_ref, lsm_p1_ref, lsm_p2_ref, fused_ref,
            buf_ref, s1_ref, s2_ref, ob_ref, sem):
    ph = pl.program_id(0)
    b = pl.program_id(1)
    i = pl.program_id(2)
    t = ph * (4 * NBLK) + b * NBLK + i
    a_refs = (a1_ref, p1_ref, a2_ref, p2_ref)

    def start(u, slot):
        row = (u % NBLK) * BM
        bu = (u // NBLK) % 4
        for bb in range(4):
            @pl.when(bu == bb)
            def _(bb=bb):
                pltpu.make_async_copy(
                    a_refs[bb].at[pl.ds(row, BM), :],
                    buf_ref.at[slot], sem.at[slot]).start()

    @pl.when(t == 0)
    def _warmup():
        for u in range(D):
            pltpu.make_async_copy(
                a_refs[0].at[pl.ds(u * BM, BM), :],
                buf_ref.at[u], sem.at[u]).start()
        for p in range(4):
            s1_ref[p] = jnp.dot(x_ref[...], w1_ref[p], **_DOT).astype(_BF)

    slot = t % D
    pltpu.make_async_copy(
        a_refs[0].at[pl.ds(0, BM), :], buf_ref.at[slot], sem.at[slot]).wait()
    panel = buf_ref[slot].astype(_BF)

    @pl.when(ph == 0)
    def _layer1():
        z = jnp.dot(panel, s1_ref[b], **_DOT)
        h = jnp.tanh(z + b1_ref[b, 0:1, :])
        s2_ref[b, pl.ds(i * BM, BM), :] = jnp.dot(
            h, w2_ref[b], **_DOT).astype(_BF)

    @pl.when(ph == 1)
    def _layer2():
        z = jnp.dot(panel, s2_ref[b], **_DOT)
        h2 = jnp.tanh(z + b2_ref[b, 0:1, :])
        ob_ref[b, pl.ds(i * BM, BM), :] = (
            jnp.dot(h2, wl_ref[b], **_DOT) + bl_ref[b, 0:1, :])

    @pl.when((ph == 1) & (b == 3))
    def _heads():
        rows = pl.ds(i * BM, BM)
        ob_a1 = ob_ref[0, rows, :]
        ob_p1 = ob_ref[1, rows, :]
        ob_a2 = ob_ref[2, rows, :]
        ob_p2 = ob_ref[3, rows, :]
        cat = jnp.concatenate([ob_a1, ob_a2], axis=1)
        fused = jnp.dot(cat, wf_ref[...], **_DOT) + bf_ref[0:1, :]

        def lsm(z):
            m = jnp.max(z, axis=1, keepdims=True)
            e = z - m
            return e - jnp.log(jnp.sum(jnp.exp(e), axis=1, keepdims=True))

        lsm_f_ref[...] = lsm(fused)
        lsm_p1_ref[...] = lsm(ob_p1)
        lsm_p2_ref[...] = lsm(ob_p2)
        fused_ref[...] = fused

    @pl.when(t + D < NSTEP)
    def _prefetch():
        start(t + D, slot)


def kernel(x, A1, P1, A2, P2,
           W1_A1, b1_A1, W2_A1, b2_A1, Wl_A1, bl_A1,
           W1_A2, b1_A2, W2_A2, b2_A2, Wl_A2, bl_A2,
           W1_P1, b1_P1, W2_P1, b2_P1, Wl_P1, bl_P1,
           W1_P2, b1_P2, W2_P2, b2_P2, Wl_P2, bl_P2,
           Wf, bf):
    f32 = jnp.float32
    stack = lambda *ws: jnp.stack(ws)
    W1s = stack(W1_A1, W1_P1, W1_A2, W1_P2)
    b1s = jnp.broadcast_to(
        stack(b1_A1, b1_P1, b1_A2, b1_P2)[:, None, :], (4, 8, 32))
    W2s = stack(W2_A1, W2_P1, W2_A2, W2_P2)
    b2s = jnp.broadcast_to(
        stack(b2_A1, b2_P1, b2_A2, b2_P2)[:, None, :], (4, 8, 16))
    Wls = stack(Wl_A1, Wl_P1, Wl_A2, Wl_P2)
    bls = jnp.broadcast_to(
        stack(bl_A1, bl_P1, bl_A2, bl_P2)[:, None, :], (4, 8, 8))
    bfc = jnp.broadcast_to(bf[None, :], (8, 8))

    grid = (2, 4, NBLK)
    any_spec = pl.BlockSpec(memory_space=pltpu.HBM)
    full2 = lambda r, c: pl.BlockSpec((r, c), lambda ph, b, i: (0, 0))
    full3 = lambda s: pl.BlockSpec(s, lambda ph, b, i: (0, 0, 0))
    o_spec = pl.BlockSpec((BM, 8), lambda ph, b, i: (i, 0))

    outs = pl.pallas_call(
        _kernel,
        grid=grid,
        in_specs=[full2(N, 128), any_spec, any_spec, any_spec, any_spec,
                  full3((4, 128, 32)), full3((4, 8, 32)),
                  full3((4, 32, 16)), full3((4, 8, 16)),
                  full3((4, 16, 8)), full3((4, 8, 8)),
                  full2(16, 8), full2(8, 8)],
        out_specs=[o_spec, o_spec, o_spec, o_spec],
        out_shape=[jax.ShapeDtypeStruct((N, 8), f32) for _ in range(4)],
        scratch_shapes=[pltpu.VMEM((D, BM, N), f32),
                        pltpu.VMEM((4, N, 32), _BF),
                        pltpu.VMEM((4, N, 16), _BF),
                        pltpu.VMEM((4, N, 8), f32),
                        pltpu.SemaphoreType.DMA((D,))],
        compiler_params=pltpu.CompilerParams(
            dimension_semantics=("arbitrary", "arbitrary", "arbitrary"),
            vmem_limit_bytes=64 * 1024 * 1024),
    )(x, A1, P1, A2, P2, W1s, b1s, W2s, b2s, Wls, bls, Wf, bfc)

    return tuple(outs)


# speedup vs baseline: 1.0539x; 1.0539x over previous
import jax
import jax.numpy as jnp
from jax.experimental import pallas as pl
from jax.experimental.pallas import tpu as pltpu

N = 8192
BM = 128
DEPTHS = (3, 3, 3, 3)
NSTEP = 2 * (N // BM)

_DOT = dict(precision=jax.lax.Precision.DEFAULT,
            preferred_element_type=jnp.float32)


def _dma(a_refs, bufs, sems, b, step, slot):
    row = (step % (N // BM)) * BM
    return pltpu.make_async_copy(
        a_refs[b].at[pl.ds(row, BM), :], bufs[b].at[slot], sems[b].at[slot])


def _mega_kernel(x_ref, a1_ref, p1_ref, a2_ref, p2_ref,
                 w1_ref, b1_ref, w2_ref, b2_ref, wl_ref, bl_ref,
                 wf_ref, bf_ref,
                 lsm_f_ref, lsm_p1_ref, lsm_p2_ref, fused_ref,
                 b1s_ref, b2s_ref, b3s_ref, b4s_ref, s1_ref, s2_ref,
                 sem1, sem2, sem3, sem4):
    ph = pl.program_id(0)
    i = pl.program_id(1)
    s = ph * (N // BM) + i
    a_refs = (a1_ref, p1_ref, a2_ref, p2_ref)
    bufs = (b1s_ref, b2s_ref, b3s_ref, b4s_ref)
    sems = (sem1, sem2, sem3, sem4)

    @pl.when(s == 0)
    def _warmup():
        for b in range(4):
            for t in range(DEPTHS[b]):
                _dma(a_refs, bufs, sems, b, t, t).start()

    @pl.when((ph == 0) & (i == 0))
    def _build_s1():
        s1_ref[...] = jnp.dot(x_ref[...], w1_ref[...],
                              **_DOT).astype(jnp.bfloat16)

    slots = [jax.lax.rem(s, d) for d in DEPTHS]
    for b in range(4):
        _dma(a_refs, bufs, sems, b, s, slots[b]).wait()

    @pl.when(ph == 0)
    def _layer1():
        zs = [jnp.dot(bufs[p][slots[p]].astype(jnp.bfloat16),
                      s1_ref[:, 32 * p:32 * (p + 1)], **_DOT)
              for p in range(4)]
        h = jnp.tanh(jnp.concatenate(zs, axis=1) + b1_ref[0:1, :])
        s2_ref[pl.ds(i * BM, BM), :] = jnp.dot(
            h, w2_ref[...], **_DOT).astype(jnp.bfloat16)

    @pl.when(ph == 1)
    def _layer2():
        zs = [jnp.dot(bufs[p][slots[p]].astype(jnp.bfloat16),
                      s2_ref[:, 16 * p:16 * (p + 1)], **_DOT)
              for p in range(4)]
        h2 = jnp.tanh(jnp.concatenate(zs, axis=1) + b2_ref[0:1, :])
        ob = jnp.dot(h2, wl_ref[...], **_DOT) + bl_ref[0:1, :]
        fused = jnp.dot(ob, wf_ref[...], **_DOT) + bf_ref[0:1, :]

        def lsm(z):
            m = jnp.max(z, axis=1, keepdims=True)
            e = z - m
            return e - jnp.log(jnp.sum(jnp.exp(e), axis=1, keepdims=True))

        lsm_f_ref[...] = lsm(fused)
        lsm_p1_ref[...] = lsm(ob[:, 8:16])
        lsm_p2_ref[...] = lsm(ob[:, 24:32])
        fused_ref[...] = fused

    for b in range(4):
        @pl.when(s + DEPTHS[b] < NSTEP)
        def _prefetch(b=b):
            _dma(a_refs, bufs, sems, b, s + DEPTHS[b], slots[b]).start()


def kernel(x, A1, P1, A2, P2,
           W1_A1, b1_A1, W2_A1, b2_A1, Wl_A1, bl_A1,
           W1_A2, b1_A2, W2_A2, b2_A2, Wl_A2, bl_A2,
           W1_P1, b1_P1, W2_P1, b2_P1, Wl_P1, bl_P1,
           W1_P2, b1_P2, W2_P2, b2_P2, Wl_P2, bl_P2,
           Wf, bf):
    f32 = jnp.float32
    W1c = jnp.concatenate([W1_A1, W1_P1, W1_A2, W1_P2], axis=1)
    b1c = jnp.broadcast_to(
        jnp.concatenate([b1_A1, b1_P1, b1_A2, b1_P2])[None, :], (8, 128))
    W2bd = jax.scipy.linalg.block_diag(W2_A1, W2_P1, W2_A2, W2_P2)
    b2c = jnp.broadcast_to(
        jnp.concatenate([b2_A1, b2_P1, b2_A2, b2_P2])[None, :], (8, 64))
    Wlbd = jax.scipy.linalg.block_diag(Wl_A1, Wl_P1, Wl_A2, Wl_P2)
    blc = jnp.broadcast_to(
        jnp.concatenate([bl_A1, bl_P1, bl_A2, bl_P2])[None, :], (8, 32))
    Wg = jnp.zeros((32, 8), f32).at[0:8].set(Wf[0:8]).at[16:24].set(Wf[8:16])
    bfc = jnp.broadcast_to(bf[None, :], (8, 8))

    grid = (2, N // BM)
    any_spec = pl.BlockSpec(memory_space=pltpu.HBM)
    full = lambda r, c: pl.BlockSpec((r, c), lambda ph, i: (0, 0))
    o_spec = pl.BlockSpec((BM, 8), lambda ph, i: (i, 0))

    outs = pl.pallas_call(
        _mega_kernel,
        grid=grid,
        in_specs=[full(N, 128), any_spec, any_spec, any_spec, any_spec,
                  full(128, 128), full(8, 128), full(128, 64), full(8, 64),
                  full(64, 32), full(8, 32), full(32, 8), full(8, 8)],
        out_specs=[o_spec, o_spec, o_spec, o_spec],
        out_shape=[jax.ShapeDtypeStruct((N, 8), f32) for _ in range(4)],
        scratch_shapes=[pltpu.VMEM((d, BM, N), f32) for d in DEPTHS]
                       + [pltpu.VMEM((N, 128), jnp.bfloat16),
                          pltpu.VMEM((N, 64), jnp.bfloat16)]
                       + [pltpu.SemaphoreType.DMA((d,)) for d in DEPTHS],
        compiler_params=pltpu.CompilerParams(
            dimension_semantics=("arbitrary", "arbitrary"),
            vmem_limit_bytes=62 * 1024 * 1024),
    )(x, A1, P1, A2, P2, W1c, b1c, W2bd, b2c, Wlbd, blc, Wg, bfc)

    return tuple(outs)
